# baseline (speedup 1.0000x reference)
import functools

import jax
import jax.numpy as jnp
from jax import lax
from jax.experimental import pallas as pl
from jax.experimental.pallas import tpu as pltpu
from jax.experimental.pallas import tpu_sc as plsc

VX, VY = 1.0, 1.0
XMIN, YMIN, ZMIN = 0.0, -39.68, -3.0
XMAX, YMAX, ZMAX = 69.12, 39.68, 1.0
NX, NY = 69, 79
NP = NX * NY
MAXPTS = 16
COUT = 64

BB = 4
NPTS = 120000
CH = 6000
NCH = NPTS // CH
VECS = CH // 16

RNG = 1024
NRNG = 6
PPAD = RNG * NRNG


def _sc_body(pts_hbm, buf_hbm, counts_hbm,
             ptA, ptB, counter, valx, valy, valz, semA, semB):
    c = lax.axis_index("c")
    s = lax.axis_index("s")
    b = c * 2 + s // 8
    j = s % 8
    base_p = j * RNG
    pt_base = b * (NPTS * 4)
    cnt_base = b * PPAD + base_p

    zero16 = jnp.zeros((16,), jnp.int32)
    iota4 = lax.iota(jnp.int32, 16) * 4
    active = j < NRNG

    def _zero_counter(k, _):
        counter[pl.ds(k * 16, 16)] = zero16
        return 0
    lax.fori_loop(0, RNG // 16, _zero_counter, 0)

    cal, _ = plsc.scan_count(zero16)
    cbase = jnp.min(cal)

    def _start(chunk, pt_st, sem):
        off = pt_base + chunk * (CH * 4)
        pltpu.async_copy(pts_hbm.at[pl.ds(off, CH * 4)], pt_st, sem)

    def _drain(pt_st, sem):
        pltpu.make_async_copy(pts_hbm.at[pl.ds(0, CH * 4)], pt_st,
                              sem).wait()

    def _process(pt_st):
        def _fetch(v):
            idxx = iota4 + v * 64
            xv = plsc.load_gather(pt_st, [idxx])
            yv = plsc.load_gather(pt_st, [idxx + 1])
            zv = plsc.load_gather(pt_st, [idxx + 2])
            tx = (xv - XMIN) / VX
            ty = (yv - YMIN) / VY
            ix = tx.astype(jnp.int32)
            iy = ty.astype(jnp.int32)
            valid = ((tx >= 0.0) & (tx < float(NX)) & (ty >= 0.0)
                     & (ty < float(NY)) & (zv >= ZMIN) & (zv < ZMAX))
            pid = jnp.where(valid, iy * NX + ix, NP)
            ploc = pid - base_p
            inr = (ploc >= 0) & (ploc < RNG)
            psafe = jnp.where(inr, ploc, 0)
            cnt, last = plsc.scan_count(pid, inr)
            return (psafe, inr, cnt, last, xv, yv, zv)

        def _commit(carry):
            psafe, inr, cnt, last, xv, yv, zv = carry
            c0 = plsc.load_gather(counter, [psafe], mask=inr)
            rank = c0 + cnt - cbase
            keep = inr & (rank < MAXPTS)
            rsafe = jnp.where(keep, rank, 0)
            plsc.store_scatter(valx, [rsafe, psafe], xv, mask=keep)
            plsc.store_scatter(valy, [rsafe, psafe], yv, mask=keep)
            plsc.store_scatter(valz, [rsafe, psafe], zv, mask=keep)
            plsc.store_scatter(counter, [psafe], c0 + cnt + 1 - cbase,
                               mask=last & inr)

        def _vec(v, carry):
            nxt = _fetch(v + 1)
            _commit(carry)
            return nxt
        _commit(lax.fori_loop(0, VECS - 1, _vec, _fetch(0)))

    @pl.when(active)
    def _run():
        _start(0, ptA, semA)

        def _pair(i, _):
            _drain(ptA, semA)
            _start(2 * i + 1, ptB, semB)
            _process(ptA)
            _drain(ptB, semB)

            @pl.when(i < NCH // 2 - 1)
            def _():
                _start(2 * i + 2, ptA, semA)
            _process(ptB)
            return 0
        lax.fori_loop(0, NCH // 2, _pair, 0)

        def _cap(k, _):
            v = counter[pl.ds(k * 16, 16)]
            counter[pl.ds(k * 16, 16)] = jnp.minimum(v, MAXPTS)
            return 0
        lax.fori_loop(0, RNG // 16, _cap, 0)
        pltpu.sync_copy(counter, counts_hbm.at[pl.ds(cnt_base, RNG)])

        pltpu.sync_copy(valx, buf_hbm.at[b, 0, :, pl.ds(base_p, RNG)])
        pltpu.sync_copy(valy, buf_hbm.at[b, 1, :, pl.ds(base_p, RNG)])
        pltpu.sync_copy(valz, buf_hbm.at[b, 2, :, pl.ds(base_p, RNG)])


def _sc_scatter(ptsf):
    mesh = plsc.VectorSubcoreMesh(core_axis_name="c", subcore_axis_name="s")
    f = functools.partial(
        pl.kernel, mesh=mesh,
        out_type=(
            jax.ShapeDtypeStruct((BB, 3, MAXPTS, PPAD), jnp.float32),
            jax.ShapeDtypeStruct((BB * PPAD,), jnp.int32),
        ),
        scratch_types=[
            pltpu.VMEM((CH * 4,), jnp.float32),
            pltpu.VMEM((CH * 4,), jnp.float32),
            pltpu.VMEM((RNG,), jnp.int32),
            pltpu.VMEM((MAXPTS, RNG), jnp.float32),
            pltpu.VMEM((MAXPTS, RNG), jnp.float32),
            pltpu.VMEM((MAXPTS, RNG), jnp.float32),
            pltpu.SemaphoreType.DMA,
            pltpu.SemaphoreType.DMA,
        ],
        compiler_params=pltpu.CompilerParams(needs_layout_passes=False),
    )(_sc_body)
    return f(ptsf)


def _enc_body(buf_ref, cnt_ref, xc_ref, yc_ref, w_ref, b_ref, gam_ref,
              bet_ref, mu_ref, var_ref, out_ref):
    x = buf_ref[0, 0]
    y = buf_ref[0, 1]
    z = buf_ref[0, 2]
    cnt2 = cnt_ref[0, 0]
    slot = jax.lax.broadcasted_iota(jnp.int32, (MAXPTS, PPAD), 0)
    mask = slot < cnt2
    cntf = jnp.maximum(cnt2, 1).astype(jnp.float32)
    mx = jnp.sum(jnp.where(mask, x, 0.0), 0, keepdims=True) / cntf
    my = jnp.sum(jnp.where(mask, y, 0.0), 0, keepdims=True) / cntf
    mz = jnp.sum(jnp.where(mask, z, 0.0), 0, keepdims=True) / cntf
    xc = xc_ref[0]
    yc = yc_ref[0]

    sc_ = gam_ref[0] / jnp.sqrt(var_ref[0] + 1e-5)
    w0 = w_ref[0, 0] * sc_
    w1 = w_ref[0, 1] * sc_
    w2 = w_ref[0, 2] * sc_
    w3 = w_ref[0, 3] * sc_
    w4 = w_ref[0, 4] * sc_
    w5 = w_ref[0, 5] * sc_
    w6 = w_ref[0, 6] * sc_
    w7 = w_ref[0, 7] * sc_
    c0 = b_ref[0] * sc_ + bet_ref[0] - mu_ref[0] * sc_
    a0 = w0 + w3 + w6
    a1 = w1 + w4 + w7
    a2 = w2 + w5

    w6m = jnp.concatenate([-w3, -w4, -w5, -w6, -w7, c0], axis=1)
    ones = jnp.ones((1, PPAD), jnp.float32)
    m6 = jnp.concatenate([mx, my, mz, xc, yc, ones], axis=0)
    cc = jnp.dot(w6m, m6, precision=lax.Precision.HIGHEST)

    neg = jnp.float32(-1e9)
    m = jnp.full((COUT, PPAD), neg, jnp.float32)
    for i in range(MAXPTS):
        h_i = a0 * x[i:i + 1] + a1 * y[i:i + 1] + a2 * z[i:i + 1]
        m = jnp.maximum(m, jnp.where(cnt2 > i, h_i, neg))
    res = jnp.where(cnt2 > 0, jnp.maximum(m + cc, 0.0), 0.0)
    out_ref[0] = res[:, :NP]


def _encode(buf, counts4, xc3, yc3, w3d, bb, gamma, beta, mu, var):
    vspec = pl.BlockSpec((1, 8, COUT, 1), lambda bi: (0, 0, 0, 0))
    pspec = pl.BlockSpec((1, COUT, 1), lambda bi: (0, 0, 0))
    return pl.pallas_call(
        _enc_body,
        grid=(BB,),
        in_specs=[
            pl.BlockSpec((1, 3, MAXPTS, PPAD), lambda bi: (bi, 0, 0, 0)),
            pl.BlockSpec((1, 1, 1, PPAD), lambda bi: (bi, 0, 0, 0)),
            pl.BlockSpec((1, 1, PPAD), lambda bi: (0, 0, 0)),
            pl.BlockSpec((1, 1, PPAD), lambda bi: (0, 0, 0)),
            vspec, pspec, pspec, pspec, pspec, pspec,
        ],
        out_specs=pl.BlockSpec((1, COUT, NP), lambda bi: (bi, 0, 0)),
        out_shape=jax.ShapeDtypeStruct((BB, COUT, NP), jnp.float32),
    )(buf, counts4, xc3, yc3, w3d, bb, gamma, beta, mu, var)


def kernel(batched_pts, W, b, gamma, beta, bn_mean, bn_var):
    ptsf = batched_pts.reshape(BB * NPTS * 4)

    buf, counts = _sc_scatter(ptsf)

    p_idx = jnp.arange(PPAD, dtype=jnp.int32)
    xc = ((p_idx % NX).astype(jnp.float32) + 0.5) * VX + XMIN
    yc = ((p_idx // NX).astype(jnp.float32) + 0.5) * VY + YMIN

    out = _encode(buf, counts.reshape(BB, 1, 1, PPAD),
                  xc.reshape(1, 1, PPAD), yc.reshape(1, 1, PPAD),
                  W.reshape(1, 8, COUT, 1), b.reshape(1, COUT, 1),
                  gamma.reshape(1, COUT, 1), beta.reshape(1, COUT, 1),
                  bn_mean.reshape(1, COUT, 1), bn_var.reshape(1, COUT, 1))

    return out.reshape(BB, COUT, NY, NX)
